# Initial kernel scaffold; baseline (speedup 1.0000x reference)
import jax
import jax.numpy as jnp
from jax.experimental import pallas as pl


def kernel(xyz, features):
    raise NotImplementedError("write your pallas kernel here")


# speedup vs baseline: 5.8503x; 5.8503x over previous
import functools

import jax
import jax.numpy as jnp
from jax import lax
from jax.experimental import pallas as pl
from jax.experimental.pallas import tpu as pltpu
from jax.experimental.pallas import tpu_sc as plsc

_R = 128
_S = 512


def _fps_body(xt_ref, far0_ref, gidx_ref, cx_ref, cy_ref, cz_ref):
    b = pl.program_id(0)
    n = _R * _R
    x = xt_ref[0, 0]
    y = xt_ref[0, 1]
    z = xt_ref[0, 2]
    col = lax.broadcasted_iota(jnp.int32, (_R, _R), 1)
    flat = lax.broadcasted_iota(jnp.int32, (_R, _R), 0) * _R + col
    lane = lax.broadcasted_iota(jnp.int32, (1, _R), 1)
    base = b * n
    neg = jnp.float32(-jnp.inf)

    def body(t, carry):
        far, distance = carry
        r = far // _R
        c = far - r * _R
        xrow = xt_ref[0, 0, pl.ds(r, 1), :]
        yrow = xt_ref[0, 1, pl.ds(r, 1), :]
        zrow = xt_ref[0, 2, pl.ds(r, 1), :]
        sel = lane == c
        cxv = jnp.max(jnp.where(sel, xrow, neg))
        cyv = jnp.max(jnp.where(sel, yrow, neg))
        czv = jnp.max(jnp.where(sel, zrow, neg))
        gidx_ref[0, 0, t] = base + far
        cx_ref[0, 0, t] = cxv
        cy_ref[0, 0, t] = cyv
        cz_ref[0, 0, t] = czv
        dx = x - cxv
        dy = y - cyv
        dz = z - czv
        d = dx * dx + dy * dy + dz * dz
        nd = jnp.minimum(distance, d)
        m = jnp.max(nd)
        nf = jnp.min(jnp.where(nd == m, flat, n))
        return (nf, nd)

    dist0 = jnp.full((_R, _R), 1e10, jnp.float32)
    lax.fori_loop(0, _S, body, (far0_ref[b], dist0))


def _sc_geometry():
    try:
        info = plsc.get_sparse_core_info()
        return info.num_cores, info.num_subcores
    except Exception:
        return 2, 16


def _build_sc_gather(num_rows, feat_dim, nc, ns):
    nw = nc * ns
    per_w = num_rows // nw
    j_chunks = per_w // 128
    mesh = plsc.VectorSubcoreMesh(core_axis_name="c", subcore_axis_name="s")

    @functools.partial(
        pl.kernel,
        out_type=jax.ShapeDtypeStruct((num_rows, feat_dim), jnp.float32),
        mesh=mesh,
        scratch_types=[
            pltpu.VMEM((j_chunks, 128), jnp.int32),
            pltpu.VMEM((128, feat_dim), jnp.float32),
            pltpu.SemaphoreType.DMA,
        ],
    )
    def gather(table_hbm, idx_hbm, out_hbm, idx_v, rows_v, sem):
        wid = lax.axis_index("s") * nc + lax.axis_index("c")
        pltpu.sync_copy(idx_hbm.at[wid], idx_v)
        for j in range(j_chunks):
            pltpu.async_copy(table_hbm.at[idx_v.at[j]], rows_v, sem).wait()
            pltpu.sync_copy(rows_v, out_hbm.at[pl.ds(wid * per_w + j * 128, 128)])

    return gather


def kernel(xyz, features):
    B, N, _ = xyz.shape
    F = features.shape[-1]
    xt = jnp.transpose(xyz, (0, 2, 1)).reshape(B, 3, _R, _R)
    far0 = jax.random.randint(jax.random.key(1), (B,), 0, N).astype(jnp.int32)

    gidx, cx, cy, cz = pl.pallas_call(
        _fps_body,
        grid=(B,),
        in_specs=[
            pl.BlockSpec((1, 3, _R, _R), lambda b: (b, 0, 0, 0)),
            pl.BlockSpec(memory_space=pltpu.SMEM),
        ],
        out_specs=[
            pl.BlockSpec((1, 1, _S), lambda b: (b, 0, 0), memory_space=pltpu.SMEM)
        ] * 4,
        out_shape=[jax.ShapeDtypeStruct((B, 1, _S), jnp.int32)]
        + [jax.ShapeDtypeStruct((B, 1, _S), jnp.float32)] * 3,
        compiler_params=pltpu.CompilerParams(dimension_semantics=("arbitrary",)),
    )(xt, far0)

    gidx = gidx.reshape(B, _S)
    new_xyz = jnp.stack([cx.reshape(B, _S), cy.reshape(B, _S), cz.reshape(B, _S)], axis=-1)

    nc, ns = _sc_geometry()
    nw = nc * ns
    table = features.reshape(B * N, F)
    idx3 = gidx.reshape(nw, (B * _S) // nw // 128, 128)
    new_features = _build_sc_gather(B * _S, F, nc, ns)(table, idx3).reshape(B, _S, F)
    return (new_xyz, new_features)


# speedup vs baseline: 28.4393x; 4.8612x over previous
import functools

import jax
import jax.numpy as jnp
from jax import lax
from jax.experimental import pallas as pl
from jax.experimental.pallas import tpu as pltpu
from jax.experimental.pallas import tpu_sc as plsc

_R = 128
_S = 512


_BPG = 16
_N = _R * _R


def _fps_body(xt_ref, far0_ref, out_ref, dist_ref):
    pid = pl.program_id(0)
    col = lax.broadcasted_iota(jnp.int32, (_BPG, _N), 1)
    bb = (lax.broadcasted_iota(jnp.int32, (_BPG, 1), 0) + pid * _BPG) * _N
    neg = jnp.float32(-jnp.inf)
    dist_ref[...] = jnp.full((_BPG, _N), 1e10, jnp.float32)
    x = xt_ref[0]
    y = xt_ref[1]
    z = xt_ref[2]

    def body(t, fidx):
        sel = col == fidx
        cx = jnp.max(jnp.where(sel, x, neg), axis=1, keepdims=True)
        cy = jnp.max(jnp.where(sel, y, neg), axis=1, keepdims=True)
        cz = jnp.max(jnp.where(sel, z, neg), axis=1, keepdims=True)
        gv = lax.bitcast_convert_type(fidx + bb, jnp.float32)
        row = jnp.concatenate([gv, cx, cy, cz], axis=1).reshape(1, _BPG, 4)
        out_ref[pl.ds(t, 1), :, :] = row
        dx = x - cx
        dy = y - cy
        dz = z - cz
        d = dx * dx + dy * dy + dz * dz
        nd = jnp.minimum(dist_ref[...], d)
        dist_ref[...] = nd
        m = jnp.max(nd, axis=1, keepdims=True)
        return jnp.min(jnp.where(nd == m, col, _N), axis=1, keepdims=True)

    lax.fori_loop(0, _S, body, far0_ref[...])


def _sc_geometry():
    try:
        info = plsc.get_sparse_core_info()
        return info.num_cores, info.num_subcores
    except Exception:
        return 2, 16


def _build_sc_gather(num_rows, feat_dim, nc, ns):
    nw = nc * ns
    per_w = num_rows // nw
    j_chunks = per_w // 128
    mesh = plsc.VectorSubcoreMesh(core_axis_name="c", subcore_axis_name="s")

    @functools.partial(
        pl.kernel,
        out_type=jax.ShapeDtypeStruct((num_rows, feat_dim), jnp.float32),
        mesh=mesh,
        scratch_types=[
            pltpu.VMEM((j_chunks, 128), jnp.int32),
            pltpu.VMEM((128, feat_dim), jnp.float32),
            pltpu.SemaphoreType.DMA,
        ],
    )
    def gather(table_hbm, idx_hbm, out_hbm, idx_v, rows_v, sem):
        wid = lax.axis_index("s") * nc + lax.axis_index("c")
        pltpu.sync_copy(idx_hbm.at[wid], idx_v)
        for j in range(j_chunks):
            pltpu.async_copy(table_hbm.at[idx_v.at[j]], rows_v, sem).wait()
            pltpu.sync_copy(rows_v, out_hbm.at[pl.ds(wid * per_w + j * 128, 128)])

    return gather


def kernel(xyz, features):
    B, N, _ = xyz.shape
    F = features.shape[-1]
    xt = jnp.transpose(xyz, (2, 0, 1))
    far0 = jax.random.randint(jax.random.key(1), (B,), 0, N).astype(jnp.int32)[:, None]

    out = pl.pallas_call(
        _fps_body,
        grid=(B // _BPG,),
        in_specs=[
            pl.BlockSpec((3, _BPG, _N), lambda b: (0, b, 0)),
            pl.BlockSpec((_BPG, 1), lambda b: (b, 0)),
        ],
        out_specs=pl.BlockSpec((_S, _BPG, 4), lambda b: (0, b, 0)),
        out_shape=jax.ShapeDtypeStruct((_S, B, 4), jnp.float32),
        scratch_shapes=[pltpu.VMEM((_BPG, _N), jnp.float32)],
        compiler_params=pltpu.CompilerParams(dimension_semantics=("parallel",)),
    )(xt, far0)

    gidx = lax.bitcast_convert_type(out[..., 0], jnp.int32).T
    new_xyz = jnp.transpose(out[..., 1:4], (1, 0, 2))

    nc, ns = _sc_geometry()
    nw = nc * ns
    table = features.reshape(B * N, F)
    idx3 = gidx.reshape(nw, (B * _S) // nw // 128, 128)
    new_features = _build_sc_gather(B * _S, F, nc, ns)(table, idx3).reshape(B, _S, F)
    return (new_xyz, new_features)
